# baseline (speedup 1.0000x reference)
import functools

import jax
import jax.numpy as jnp
from jax import lax
from jax.experimental import pallas as pl
from jax.experimental.pallas import tpu as pltpu
from jax.experimental.pallas import tpu_sc as plsc

N = 10000
E = 320000
D = 128
C = 40

NC = 2
NS = 16
NW = NC * NS
EPW = E // NW
K = 80
NCHUNK = EPW // K
M = 5
NBLK = NCHUNK // M
NP = 10240
RPT = NP // NS
ZROWS = 16

_f32 = jnp.float32


def _spmm_sc(h, eidx, adjr):
    mesh = plsc.VectorSubcoreMesh(core_axis_name="c", subcore_axis_name="s")

    @functools.partial(
        pl.kernel,
        mesh=mesh,
        out_type=jax.ShapeDtypeStruct((NC, NP, D), _f32),
        scratch_types=[
            pltpu.VMEM_SHARED((NP, D), _f32),
            pltpu.VMEM((M, K), jnp.int32),
            pltpu.VMEM((M, K), jnp.int32),
            pltpu.VMEM((M, K), jnp.int32),
            pltpu.VMEM((M, K), jnp.int32),
            pltpu.VMEM((M, K), _f32),
            pltpu.VMEM((M, K), _f32),
            pltpu.VMEM((K, D), _f32),
            pltpu.VMEM((K, D), _f32),
            pltpu.VMEM((K, D), _f32),
            pltpu.VMEM((K, D), _f32),
            pltpu.VMEM((ZROWS, D), _f32),
            pltpu.SemaphoreType.DMA,
            pltpu.SemaphoreType.DMA,
            pltpu.SemaphoreType.DMA,
            pltpu.SemaphoreType.DMA,
            pltpu.SemaphoreType.DMA,
            pltpu.SemaphoreType.DMA,
            pltpu.SemaphoreType.DMA,
            pltpu.SemaphoreType.DMA,
            pltpu.SemaphoreType.DMA,
            pltpu.SemaphoreType.DMA,
        ],
    )
    def spmm(h_hbm, eidx_hbm, adj_hbm, out_hbm,
             acc_sh, cx0, cx1, rx0, rx1, ax0, ax1,
             buf0, buf1, buf2, buf3, zbuf,
             sg0, sg1, sg2, sg3, ss0, ss1, ss2, ss3, sx, sz):
        cid = lax.axis_index("c")
        sid = lax.axis_index("s")
        wid = cid * NS + sid
        bufs = (buf0, buf1, buf2, buf3)
        cxs = (cx0, cx1)
        rxs = (rx0, rx1)
        axs = (ax0, ax1)
        sgs = (sg0, sg1, sg2, sg3)
        sss = (ss0, ss1, ss2, ss3)

        def issue_blk(bk, s):
            bk = jnp.minimum(bk, NBLK - 1)
            pltpu.async_copy(eidx_hbm.at[1, wid, bk], cxs[s], sx)
            pltpu.async_copy(eidx_hbm.at[0, wid, bk], rxs[s], sx)
            pltpu.async_copy(adj_hbm.at[wid, bk], axs[s], sx)

        def wait_blk(bk, s):
            bk = jnp.minimum(bk, NBLK - 1)
            pltpu.make_async_copy(eidx_hbm.at[1, wid, bk], cxs[s], sx).wait()
            pltpu.make_async_copy(eidx_hbm.at[0, wid, bk], rxs[s], sx).wait()
            pltpu.make_async_copy(adj_hbm.at[wid, bk], axs[s], sx).wait()

        issue_blk(0, 0)
        zero16 = jnp.zeros((16,), _f32)

        def _zrow(r, _):
            for j in range(D // 16):
                zbuf[r, pl.ds(j * 16, 16)] = zero16
            return 0

        lax.fori_loop(0, ZROWS, _zrow, 0)
        zdsts = [acc_sh.at[pl.ds(sid * RPT + t * ZROWS, ZROWS)]
                 for t in range(RPT // ZROWS)]
        for zd in zdsts:
            pltpu.async_copy(zbuf, zd, sz)

        def start_gather(g, m, s, b):
            pltpu.async_copy(h_hbm.at[cxs[s].at[m]], bufs[b], sgs[b])

        def wait_gather(g, m, s, b):
            del g
            pltpu.make_async_copy(h_hbm.at[cxs[s].at[m]],
                                  bufs[b], sgs[b]).wait()

        def start_scatter(m, s, b):
            pltpu.async_copy(bufs[b], acc_sh.at[rxs[s].at[m]],
                             sss[b], add=True)

        def wait_scatter(m, s, b):
            pltpu.make_async_copy(bufs[b], acc_sh.at[rxs[s].at[m]],
                                  sss[b]).wait()

        def scale(m, s, b):
            buf = bufs[b]
            ax = axs[s]

            def _scale(q, _):
                av = ax[m, pl.ds(q * 16, 16)]
                for l in range(16):
                    e = q * 16 + l
                    sp = jnp.full((16,), av[l], _f32)
                    for j in range(D // 16):
                        sl = pl.ds(j * 16, 16)
                        buf[e, sl] = buf[e, sl] * sp
                return 0

            lax.fori_loop(0, K // 16, _scale, 0)

        def phase_step(g, ph, do_ws=True, do_gather=True):
            b = ph % 4
            bg = (ph + 2) % 4
            m = ph % 5
            s = (ph // 5) % 2
            sn = (s + 1) % 2
            m2 = (ph - 2) % 5
            s2 = (((ph - 2) % 20) // 5) % 2
            mg = (ph + 2) % 5
            sg = (((ph + 2) % 20) // 5) % 2
            if do_ws:
                wait_scatter(m2, s2, bg)
            if m == 1:
                issue_blk(g // 5 + 1, sn)
            if do_gather:
                if mg == 0:
                    wait_blk((g + 2) // 5, sg)
                start_gather(g + 2, mg, sg, bg)
            wait_gather(g, m, s, b)
            scale(m, s, b)
            start_scatter(m, s, b)

        wait_blk(0, 0)
        start_gather(0, 0, 0, 0)
        start_gather(1, 1, 0, 1)
        for zd in zdsts:
            pltpu.make_async_copy(zbuf, zd, sz).wait()
        plsc.subcore_barrier()
        phase_step(0, 0, do_ws=False)
        phase_step(1, 1, do_ws=False)

        def _body(i, _):
            g0 = 2 + 20 * i
            for r in range(20):
                phase_step(g0 + r, (2 + r) % 20)
            return 0

        lax.fori_loop(0, (NCHUNK - 5) // 20, _body, 0)

        phase_step(NCHUNK - 3, (NCHUNK - 3) % 20)
        phase_step(NCHUNK - 2, (NCHUNK - 2) % 20, do_gather=False)
        phase_step(NCHUNK - 1, (NCHUNK - 1) % 20, do_gather=False)
        wait_scatter((NCHUNK - 2) % 5, (((NCHUNK - 2) % 20) // 5) % 2,
                     (NCHUNK - 2) % 4)
        wait_scatter((NCHUNK - 1) % 5, (((NCHUNK - 1) % 20) // 5) % 2,
                     (NCHUNK - 1) % 4)
        wait_blk(NBLK, 1)
        plsc.subcore_barrier()

        sl = pl.ds(sid * RPT, RPT)
        pltpu.sync_copy(acc_sh.at[sl], out_hbm.at[cid, sl])

    return spmm(h, eidx, adjr)


BN = 10000


def _mm_body(x_ref, w_ref, o_ref):
    o_ref[...] = jnp.dot(x_ref[...], w_ref[...], preferred_element_type=_f32)


def _mm(x, w):
    return pl.pallas_call(
        _mm_body,
        grid=(N // BN,),
        in_specs=[
            pl.BlockSpec((BN, D), lambda i: (i, 0)),
            pl.BlockSpec((D, D), lambda i: (0, 0)),
        ],
        out_specs=pl.BlockSpec((BN, D), lambda i: (i, 0)),
        out_shape=jax.ShapeDtypeStruct((N, D), _f32),
    )(x, w)


def _relu_mm_body(a_ref, b_ref, w_ref, o_ref):
    a = jnp.maximum(a_ref[0] + b_ref[0], 0.0)
    o_ref[...] = jnp.dot(a, w_ref[...], preferred_element_type=_f32)


def _relu_mm(p, w):
    return pl.pallas_call(
        _relu_mm_body,
        grid=(N // BN,),
        in_specs=[
            pl.BlockSpec((1, BN, D), lambda i: (0, i, 0)),
            pl.BlockSpec((1, BN, D), lambda i: (1, i, 0)),
            pl.BlockSpec((D, D), lambda i: (0, 0)),
        ],
        out_specs=pl.BlockSpec((BN, D), lambda i: (i, 0)),
        out_shape=jax.ShapeDtypeStruct((N, D), _f32),
    )(p, p, w)


def _head_body(a_ref, b_ref, wc_ref, bc_ref, out_ref, probs_ref):
    a = jnp.maximum(a_ref[0] + b_ref[0], 0.0)
    sq = jnp.sum(a * a, axis=1, keepdims=True)
    out = a * lax.rsqrt(jnp.maximum(sq, 1e-12))
    out_ref[...] = out
    logits = jnp.dot(out, wc_ref[...], preferred_element_type=_f32) + bc_ref[...]
    m = jnp.max(logits, axis=1, keepdims=True)
    p = jnp.exp(logits - m)
    probs_ref[...] = p / jnp.sum(p, axis=1, keepdims=True)


def _head(p, wc, bc2d):
    return pl.pallas_call(
        _head_body,
        grid=(N // BN,),
        in_specs=[
            pl.BlockSpec((1, BN, D), lambda i: (0, i, 0)),
            pl.BlockSpec((1, BN, D), lambda i: (1, i, 0)),
            pl.BlockSpec((D, C), lambda i: (0, 0)),
            pl.BlockSpec((1, C), lambda i: (0, 0)),
        ],
        out_specs=[
            pl.BlockSpec((BN, D), lambda i: (i, 0)),
            pl.BlockSpec((BN, C), lambda i: (i, 0)),
        ],
        out_shape=[
            jax.ShapeDtypeStruct((N, D), _f32),
            jax.ShapeDtypeStruct((N, C), _f32),
        ],
    )(p, p, wc, bc2d)


def kernel(input_embed, edge_index, adj_values, W0, W1, Wc, bc):
    eidx = edge_index.reshape(2, NW, NBLK, M, K)
    adjr = adj_values.reshape(NW, NBLK, M, K)
    bc2d = bc.reshape(1, C)

    h0 = _mm(input_embed, W0)
    p0 = _spmm_sc(h0, eidx, adjr)
    h1 = _relu_mm(p0, W1)
    p1 = _spmm_sc(h1, eidx, adjr)
    output, probs = _head(p1, Wc, bc2d)
    return (output, probs)


# speedup vs baseline: 1.0098x; 1.0098x over previous
import functools

import jax
import jax.numpy as jnp
from jax import lax
from jax.experimental import pallas as pl
from jax.experimental.pallas import tpu as pltpu
from jax.experimental.pallas import tpu_sc as plsc

N = 10000
E = 320000
D = 128
C = 40

NC = 2
NS = 16
NW = NC * NS
EPW = E // NW
K = 80
NCHUNK = EPW // K
M = 5
NBLK = NCHUNK // M
NP = 10240
RPT = NP // NS
ZROWS = 16

_f32 = jnp.float32


def _spmm_sc(h, eidx, adjr):
    mesh = plsc.VectorSubcoreMesh(core_axis_name="c", subcore_axis_name="s")

    @functools.partial(
        pl.kernel,
        mesh=mesh,
        out_type=jax.ShapeDtypeStruct((NC, NP, D), _f32),
        scratch_types=[
            pltpu.VMEM_SHARED((NP, D), _f32),
            pltpu.VMEM((M, K), jnp.int32),
            pltpu.VMEM((M, K), jnp.int32),
            pltpu.VMEM((M, K), jnp.int32),
            pltpu.VMEM((M, K), jnp.int32),
            pltpu.VMEM((M, K), _f32),
            pltpu.VMEM((M, K), _f32),
            pltpu.VMEM((K, D), _f32),
            pltpu.VMEM((K, D), _f32),
            pltpu.VMEM((K, D), _f32),
            pltpu.VMEM((K, D), _f32),
            pltpu.VMEM((ZROWS, D), _f32),
            pltpu.SemaphoreType.DMA,
            pltpu.SemaphoreType.DMA,
            pltpu.SemaphoreType.DMA,
            pltpu.SemaphoreType.DMA,
            pltpu.SemaphoreType.DMA,
            pltpu.SemaphoreType.DMA,
            pltpu.SemaphoreType.DMA,
            pltpu.SemaphoreType.DMA,
            pltpu.SemaphoreType.DMA,
            pltpu.SemaphoreType.DMA,
        ],
    )
    def spmm(h_hbm, eidx_hbm, adj_hbm, out_hbm,
             acc_sh, cx0, cx1, rx0, rx1, ax0, ax1,
             buf0, buf1, buf2, buf3, zbuf,
             sg0, sg1, sg2, sg3, ss0, ss1, ss2, ss3, sx, sz):
        cid = lax.axis_index("c")
        sid = lax.axis_index("s")
        wid = cid * NS + sid
        bufs = (buf0, buf1, buf2, buf3)
        cxs = (cx0, cx1)
        rxs = (rx0, rx1)
        axs = (ax0, ax1)
        sgs = (sg0, sg1, sg2, sg3)
        sss = (ss0, ss1, ss2, ss3)

        def issue_blk(bk, s):
            bk = jnp.minimum(bk, NBLK - 1)
            pltpu.async_copy(eidx_hbm.at[1, wid, bk], cxs[s], sx)
            pltpu.async_copy(eidx_hbm.at[0, wid, bk], rxs[s], sx)
            pltpu.async_copy(adj_hbm.at[wid, bk], axs[s], sx)

        def wait_blk(bk, s):
            bk = jnp.minimum(bk, NBLK - 1)
            pltpu.make_async_copy(eidx_hbm.at[1, wid, bk], cxs[s], sx).wait()
            pltpu.make_async_copy(eidx_hbm.at[0, wid, bk], rxs[s], sx).wait()
            pltpu.make_async_copy(adj_hbm.at[wid, bk], axs[s], sx).wait()

        issue_blk(0, 0)
        zero16 = jnp.zeros((16,), _f32)

        def _zrow(r, _):
            for j in range(D // 16):
                zbuf[r, pl.ds(j * 16, 16)] = zero16
            return 0

        lax.fori_loop(0, ZROWS, _zrow, 0)
        zdsts = [acc_sh.at[pl.ds(sid * RPT + t * ZROWS, ZROWS)]
                 for t in range(RPT // ZROWS)]
        for zd in zdsts:
            pltpu.async_copy(zbuf, zd, sz)

        def start_gather(g, m, s, b):
            pltpu.async_copy(h_hbm.at[cxs[s].at[m]], bufs[b], sgs[b])

        def wait_gather(g, m, s, b):
            del g
            pltpu.make_async_copy(h_hbm.at[cxs[s].at[m]],
                                  bufs[b], sgs[b]).wait()

        def start_scatter(m, s, b):
            pltpu.async_copy(bufs[b], acc_sh.at[rxs[s].at[m]],
                             sss[b], add=True)

        def wait_scatter(m, s, b):
            pltpu.make_async_copy(bufs[b], acc_sh.at[rxs[s].at[m]],
                                  sss[b]).wait()

        def scale(m, s, b):
            buf = bufs[b]
            ax = axs[s]

            def _scale(q, _):
                av = ax[m, pl.ds(q * 16, 16)]
                for l in range(16):
                    e = q * 16 + l
                    sp = jnp.full((16,), av[l], _f32)
                    for j in range(D // 16):
                        sl = pl.ds(j * 16, 16)
                        buf[e, sl] = buf[e, sl] * sp
                return 0

            lax.fori_loop(0, K // 16, _scale, 0)

        def phase_step(g, ph, do_ws=True, do_gather=True):
            b = ph % 4
            bg = (ph + 2) % 4
            m = ph % 5
            s = (ph // 5) % 2
            sn = (s + 1) % 2
            m2 = (ph - 2) % 5
            s2 = (((ph - 2) % 20) // 5) % 2
            mg = (ph + 2) % 5
            sg = (((ph + 2) % 20) // 5) % 2
            if do_ws:
                wait_scatter(m2, s2, bg)
            if m == 1:
                issue_blk(g // 5 + 1, sn)
            if do_gather:
                if mg == 0:
                    wait_blk((g + 2) // 5, sg)
                start_gather(g + 2, mg, sg, bg)
            wait_gather(g, m, s, b)
            scale(m, s, b)
            start_scatter(m, s, b)

        wait_blk(0, 0)
        start_gather(0, 0, 0, 0)
        start_gather(1, 1, 0, 1)
        for zd in zdsts:
            pltpu.make_async_copy(zbuf, zd, sz).wait()
        plsc.subcore_barrier()
        phase_step(0, 0, do_ws=False)
        phase_step(1, 1, do_ws=False)

        def _body(i, _):
            g0 = 2 + 20 * i
            for r in range(20):
                phase_step(g0 + r, (2 + r) % 20)
            return 0

        lax.fori_loop(0, (NCHUNK - 5) // 20, _body, 0)

        phase_step(NCHUNK - 3, (NCHUNK - 3) % 20)
        phase_step(NCHUNK - 2, (NCHUNK - 2) % 20, do_gather=False)
        phase_step(NCHUNK - 1, (NCHUNK - 1) % 20, do_gather=False)
        wait_scatter((NCHUNK - 2) % 5, (((NCHUNK - 2) % 20) // 5) % 2,
                     (NCHUNK - 2) % 4)
        wait_scatter((NCHUNK - 1) % 5, (((NCHUNK - 1) % 20) // 5) % 2,
                     (NCHUNK - 1) % 4)
        wait_blk(NBLK, 1)
        plsc.subcore_barrier()

        sl = pl.ds(sid * RPT, RPT)
        pltpu.sync_copy(acc_sh.at[sl], out_hbm.at[cid, sl])

    return spmm(h, eidx, adjr)


BN = 5000


def _mm_body(x_ref, w_ref, o_ref):
    o_ref[...] = jnp.dot(x_ref[...], w_ref[...], preferred_element_type=_f32)


def _mm(x, w):
    return pl.pallas_call(
        _mm_body,
        grid=(N // BN,),
        in_specs=[
            pl.BlockSpec((BN, D), lambda i: (i, 0)),
            pl.BlockSpec((D, D), lambda i: (0, 0)),
        ],
        out_specs=pl.BlockSpec((BN, D), lambda i: (i, 0)),
        out_shape=jax.ShapeDtypeStruct((N, D), _f32),
    )(x, w)


def _relu_mm_body(a_ref, b_ref, w_ref, o_ref):
    a = jnp.maximum(a_ref[0] + b_ref[0], 0.0)
    o_ref[...] = jnp.dot(a, w_ref[...], preferred_element_type=_f32)


def _relu_mm(p, w):
    return pl.pallas_call(
        _relu_mm_body,
        grid=(N // BN,),
        in_specs=[
            pl.BlockSpec((1, BN, D), lambda i: (0, i, 0)),
            pl.BlockSpec((1, BN, D), lambda i: (1, i, 0)),
            pl.BlockSpec((D, D), lambda i: (0, 0)),
        ],
        out_specs=pl.BlockSpec((BN, D), lambda i: (i, 0)),
        out_shape=jax.ShapeDtypeStruct((N, D), _f32),
    )(p, p, w)


def _head_body(a_ref, b_ref, wc_ref, bc_ref, out_ref, probs_ref):
    a = jnp.maximum(a_ref[0] + b_ref[0], 0.0)
    sq = jnp.sum(a * a, axis=1, keepdims=True)
    out = a * lax.rsqrt(jnp.maximum(sq, 1e-12))
    out_ref[...] = out
    logits = jnp.dot(out, wc_ref[...], preferred_element_type=_f32) + bc_ref[...]
    m = jnp.max(logits, axis=1, keepdims=True)
    p = jnp.exp(logits - m)
    probs_ref[...] = p / jnp.sum(p, axis=1, keepdims=True)


def _head(p, wc, bc2d):
    return pl.pallas_call(
        _head_body,
        grid=(N // BN,),
        in_specs=[
            pl.BlockSpec((1, BN, D), lambda i: (0, i, 0)),
            pl.BlockSpec((1, BN, D), lambda i: (1, i, 0)),
            pl.BlockSpec((D, C), lambda i: (0, 0)),
            pl.BlockSpec((1, C), lambda i: (0, 0)),
        ],
        out_specs=[
            pl.BlockSpec((BN, D), lambda i: (i, 0)),
            pl.BlockSpec((BN, C), lambda i: (i, 0)),
        ],
        out_shape=[
            jax.ShapeDtypeStruct((N, D), _f32),
            jax.ShapeDtypeStruct((N, C), _f32),
        ],
    )(p, p, wc, bc2d)


def kernel(input_embed, edge_index, adj_values, W0, W1, Wc, bc):
    eidx = edge_index.reshape(2, NW, NBLK, M, K)
    adjr = adj_values.reshape(NW, NBLK, M, K)
    bc2d = bc.reshape(1, C)

    h0 = _mm(input_embed, W0)
    p0 = _spmm_sc(h0, eidx, adjr)
    h1 = _relu_mm(p0, W1)
    p1 = _spmm_sc(h1, eidx, adjr)
    output, probs = _head(p1, Wc, bc2d)
    return (output, probs)
